# baseline (speedup 1.0000x reference)
import functools

import jax
import jax.numpy as jnp
import numpy as np
from jax.experimental import pallas as pl
from jax.experimental.pallas import tpu as pltpu

NUM_BITS = 16
HIDDEN = 256
C = 1 << NUM_BITS
S = 16384

_KS0 = np.uint32(0)
_KS1 = np.uint32(42)
_KS2 = np.uint32(42 ^ 0x1BD11BDA)
_R1 = (13, 15, 26, 6)
_R2 = (17, 29, 16, 24)
_TINY = np.float32(np.finfo(np.float32).tiny)


def _rotl(x, r):
    return (x << np.uint32(r)) | (x >> np.uint32(32 - r))


def _threefry_rounds(x0, x1, rots):
    for r in rots:
        x0 = x0 + x1
        x1 = _rotl(x1, r)
        x1 = x1 ^ x0
    return x0, x1


def _random_bits(lo):
    x0 = jnp.zeros_like(lo) + _KS0
    x1 = lo + _KS1
    x0, x1 = _threefry_rounds(x0, x1, _R1)
    x0 = x0 + _KS1
    x1 = x1 + (_KS2 + np.uint32(1))
    x0, x1 = _threefry_rounds(x0, x1, _R2)
    x0 = x0 + _KS2
    x1 = x1 + (_KS0 + np.uint32(2))
    x0, x1 = _threefry_rounds(x0, x1, _R1)
    x0 = x0 + _KS0
    x1 = x1 + (_KS1 + np.uint32(3))
    x0, x1 = _threefry_rounds(x0, x1, _R2)
    x0 = x0 + _KS1
    x1 = x1 + (_KS2 + np.uint32(4))
    x0, x1 = _threefry_rounds(x0, x1, _R1)
    x0 = x0 + _KS2
    x1 = x1 + (_KS0 + np.uint32(5))
    return x0 ^ x1


def _gumbel(lo):
    bits = _random_bits(lo)
    float_bits = (bits >> np.uint32(9)) | np.uint32(0x3F800000)
    f = jax.lax.bitcast_convert_type(float_bits, jnp.float32) - np.float32(1.0)
    u = jnp.maximum(_TINY, f * (np.float32(1.0) - _TINY) + _TINY)
    return -jnp.log(-jnp.log(u))


def _logits_body(w1_ref, b1_ref, w2t_ref, b2_ref, out_ref, *, cb):
    i = pl.program_id(0)
    col = i * cb + jax.lax.broadcasted_iota(jnp.int32, (NUM_BITS, cb), 1)
    shift = (NUM_BITS - 1) - jax.lax.broadcasted_iota(
        jnp.int32, (NUM_BITS, cb), 0)
    xt = ((col >> shift) & 1).astype(jnp.float32)
    ht = jax.lax.dot_general(
        w1_ref[...], xt, (((0,), (0,)), ((), ())),
        preferred_element_type=jnp.float32)
    ht = jnp.maximum(ht + b1_ref[...], 0.0)
    et = jax.lax.dot_general(
        w2t_ref[...], ht, (((1,), (0,)), ((), ())),
        preferred_element_type=jnp.float32)
    out_ref[...] = -(et + b2_ref[...])


def _sample_body(logits_ref, out_ref, *, sb, cb):
    s_base = pl.program_id(0) * sb
    row = (s_base + jax.lax.broadcasted_iota(jnp.int32, (sb, cb), 0)
           ).astype(jnp.uint32)
    col_iota = jax.lax.broadcasted_iota(jnp.int32, (sb, cb), 1)
    nc = C // cb

    def chunk(ci, carry):
        run_max, run_idx = carry
        cidx = ci * cb + col_iota
        lo = (row << np.uint32(NUM_BITS)) | cidx.astype(jnp.uint32)
        g = _gumbel(lo)
        score = g + logits_ref[:, pl.ds(ci * cb, cb)]
        tile_max = jnp.max(score, axis=1, keepdims=True)
        cand = jnp.min(
            jnp.where(score == tile_max, cidx, jnp.int32(C)),
            axis=1, keepdims=True)
        upd = tile_max > run_max
        run_max = jnp.where(upd, tile_max, run_max)
        run_idx = jnp.where(upd, cand, run_idx)
        return run_max, run_idx

    init = (jnp.full((sb, 1), -jnp.inf, jnp.float32),
            jnp.zeros((sb, 1), jnp.int32))
    _, idx = jax.lax.fori_loop(0, nc, chunk, init)

    shift = (NUM_BITS - 1) - jax.lax.broadcasted_iota(
        jnp.int32, (sb, NUM_BITS), 1)
    out_ref[...] = (idx >> shift) & 1


@functools.partial(jax.jit, static_argnames=())
def kernel(n, W1, b1, W2, b2):
    del n
    cb_logits = 2048
    logits = pl.pallas_call(
        functools.partial(_logits_body, cb=cb_logits),
        grid=(C // cb_logits,),
        in_specs=[
            pl.BlockSpec((NUM_BITS, HIDDEN), lambda i: (0, 0)),
            pl.BlockSpec((HIDDEN, 1), lambda i: (0, 0)),
            pl.BlockSpec((1, HIDDEN), lambda i: (0, 0)),
            pl.BlockSpec((1, 1), lambda i: (0, 0)),
        ],
        out_specs=pl.BlockSpec((1, cb_logits), lambda i: (0, i)),
        out_shape=jax.ShapeDtypeStruct((1, C), jnp.float32),
    )(W1, b1.reshape(HIDDEN, 1), W2.reshape(1, HIDDEN), b2.reshape(1, 1))

    sb, cb = 8, 1024
    idx_bits = pl.pallas_call(
        functools.partial(_sample_body, sb=sb, cb=cb),
        grid=(S // sb,),
        in_specs=[pl.BlockSpec((1, C), lambda i: (0, 0))],
        out_specs=pl.BlockSpec((sb, NUM_BITS), lambda i: (i, 0)),
        out_shape=jax.ShapeDtypeStruct((S, NUM_BITS), jnp.int32),
        compiler_params=pltpu.CompilerParams(
            dimension_semantics=("parallel",)),
    )(logits)
    return idx_bits.astype(jnp.int8)
